# baseline (speedup 1.0000x reference)
import functools

import jax
import jax.numpy as jnp
from jax import lax
from jax.experimental import pallas as pl
from jax.experimental.pallas import tpu as pltpu
from jax.experimental.pallas import tpu_sc as plsc

N = 10000
NP = 10240
E = 320000
D_IN = 128
H = 64
D_OUT = 128

NC = 2
NS = 16
NW = NC * NS
CH = 128
CHN = E // CH
WCH = CHN // NW
XTRA = CHN - WCH * NW

RPS = NP // NS

_mesh = plsc.VectorSubcoreMesh(core_axis_name="c", subcore_axis_name="s")
_sc_params = pltpu.CompilerParams(use_tc_tiling_on_sc=False)


def _chunk_range(cid, sid):
    wid = sid * NC + cid
    base = wid * WCH + jnp.minimum(wid, XTRA)
    extra = wid < XTRA
    return base, extra


def _deg_body(dst_hbm, ones_hbm, zeros_hbm, out_hbm, idx_v, ones_v, acc):
    cid = lax.axis_index("c")
    sid = lax.axis_index("s")
    base, extra = _chunk_range(cid, sid)
    my = pl.ds(sid * RPS, RPS)
    pltpu.sync_copy(zeros_hbm.at[my], acc.at[my])
    pltpu.sync_copy(ones_hbm, ones_v)
    pltpu.sync_copy(dst_hbm.at[pl.ds(base, WCH)], idx_v.at[pl.ds(0, WCH)])

    @pl.when(extra)
    def _():
        pltpu.sync_copy(dst_hbm.at[pl.ds(base + WCH, 1)],
                        idx_v.at[pl.ds(WCH, 1)])

    plsc.subcore_barrier()

    @pl.loop(0, WCH)
    def _(ci):
        pltpu.sync_copy(ones_v, acc.at[idx_v.at[ci]], add=True)

    @pl.when(extra)
    def _():
        pltpu.sync_copy(ones_v, acc.at[idx_v.at[WCH]], add=True)

    plsc.subcore_barrier()
    pltpu.sync_copy(acc.at[my], out_hbm.at[cid, my])


@jax.jit
def _deg_partials(dst3, ones16, zeros16):
    k = pl.kernel(
        _deg_body,
        out_type=jax.ShapeDtypeStruct((NC, NP, 16), jnp.float32),
        mesh=_mesh,
        scratch_types=[
            pltpu.VMEM((WCH + 1, CH), jnp.int32),
            pltpu.VMEM((CH, 16), jnp.float32),
            pltpu.VMEM_SHARED((NP, 16), jnp.float32),
        ],
        compiler_params=_sc_params,
    )
    return k(dst3, ones16, zeros16)


def _mp_body(g_hbm, src_hbm, dst_hbm, zeros_hbm, out_hbm,
             sidx_v, didx_v, rows, acc, semg, sems):
    cid = lax.axis_index("c")
    sid = lax.axis_index("s")
    base, extra = _chunk_range(cid, sid)
    my = pl.ds(sid * RPS, RPS)
    pltpu.sync_copy(zeros_hbm.at[my], acc.at[my])
    pltpu.sync_copy(src_hbm.at[pl.ds(base, WCH)], sidx_v.at[pl.ds(0, WCH)])
    pltpu.sync_copy(dst_hbm.at[pl.ds(base, WCH)], didx_v.at[pl.ds(0, WCH)])

    @pl.when(extra)
    def _():
        pltpu.sync_copy(src_hbm.at[pl.ds(base + WCH, 1)],
                        sidx_v.at[pl.ds(WCH, 1)])
        pltpu.sync_copy(dst_hbm.at[pl.ds(base + WCH, 1)],
                        didx_v.at[pl.ds(WCH, 1)])

    plsc.subcore_barrier()

    for j in range(8):
        pltpu.async_copy(g_hbm.at[sidx_v.at[j]], rows[j], semg[j])

    @pl.loop(0, WCH - 14, step=8)
    def _(c):
        for j in range(8):
            pltpu.make_async_copy(g_hbm.at[sidx_v.at[c + j]],
                                  rows[j], semg[j]).wait()
            pltpu.async_copy(rows[j], acc.at[didx_v.at[c + j]], sems[j],
                             add=True)
        for j in range(8):
            pltpu.make_async_copy(rows[j], acc.at[didx_v.at[c + j]],
                                  sems[j]).wait()
            pltpu.async_copy(g_hbm.at[sidx_v.at[c + 8 + j]], rows[j], semg[j])

    _c0 = WCH - 14
    for j in range(8):
        pltpu.make_async_copy(g_hbm.at[sidx_v.at[_c0 + j]],
                              rows[j], semg[j]).wait()
        pltpu.async_copy(rows[j], acc.at[didx_v.at[_c0 + j]], sems[j],
                         add=True)
    for j in range(6):
        pltpu.make_async_copy(rows[j], acc.at[didx_v.at[_c0 + j]],
                              sems[j]).wait()
        pltpu.async_copy(g_hbm.at[sidx_v.at[_c0 + 8 + j]], rows[j], semg[j])
    for j in range(6):
        pltpu.make_async_copy(g_hbm.at[sidx_v.at[_c0 + 8 + j]],
                              rows[j], semg[j]).wait()
        pltpu.async_copy(rows[j], acc.at[didx_v.at[_c0 + 8 + j]], sems[j],
                         add=True)
    for j in range(6):
        pltpu.make_async_copy(rows[j], acc.at[didx_v.at[_c0 + 8 + j]],
                              sems[j]).wait()
    for j in range(6, 8):
        pltpu.make_async_copy(rows[j], acc.at[didx_v.at[_c0 + j]],
                              sems[j]).wait()

    @pl.when(extra)
    def _():
        pltpu.async_copy(g_hbm.at[sidx_v.at[WCH]], rows[0], semg[0]).wait()
        pltpu.sync_copy(rows[0], acc.at[didx_v.at[WCH]], add=True)

    plsc.subcore_barrier()
    pltpu.sync_copy(acc.at[my], out_hbm.at[cid, my])


def _make_mp(h):
    @jax.jit
    def mp(g, src2, dst2, zeros):
        k = pl.kernel(
            _mp_body,
            out_type=jax.ShapeDtypeStruct((NC, NP, h), jnp.float32),
            mesh=_mesh,
            scratch_types=[
                pltpu.VMEM((WCH + 1, CH), jnp.int32),
                pltpu.VMEM((WCH + 1, CH), jnp.int32),
                [pltpu.VMEM((CH, h), jnp.float32) for _ in range(8)],
                pltpu.VMEM_SHARED((NP, h), jnp.float32),
                [pltpu.SemaphoreType.DMA for _ in range(8)],
                [pltpu.SemaphoreType.DMA for _ in range(8)],
            ],
            compiler_params=_sc_params,
        )
        return k(g, src2, dst2, zeros)
    return mp


_mp64 = _make_mp(H)


_R = 2048


def _tc1a_body(x_ref, w_ref, g_ref):
    g_ref[...] = jnp.dot(x_ref[...], w_ref[...],
                         preferred_element_type=jnp.float32)


@jax.jit
def _tc1a(x, W1):
    return pl.pallas_call(
        _tc1a_body,
        grid=(NP // _R,),
        in_specs=[
            pl.BlockSpec((_R, D_IN), lambda i: (i, 0)),
            pl.BlockSpec((D_IN, H), lambda i: (0, 0)),
        ],
        out_specs=pl.BlockSpec((_R, H), lambda i: (i, 0)),
        out_shape=jax.ShapeDtypeStruct((NP, H), jnp.float32),
    )(x, W1)


def _tc1b_body(p_ref, graw_ref, g_ref, dis_ref):
    p = p_ref[...]
    deg = p[0, :, 0:1] + p[1, :, 0:1] + 1.0
    dis = lax.rsqrt(deg)
    g_ref[...] = graw_ref[...] * dis
    dis_ref[...] = dis


@jax.jit
def _tc1b(degp, graw):
    return pl.pallas_call(
        _tc1b_body,
        grid=(NP // _R,),
        in_specs=[
            pl.BlockSpec((NC, _R, 16), lambda i: (0, i, 0)),
            pl.BlockSpec((_R, H), lambda i: (i, 0)),
        ],
        out_specs=[
            pl.BlockSpec((_R, H), lambda i: (i, 0)),
            pl.BlockSpec((_R, 1), lambda i: (i, 0)),
        ],
        out_shape=[
            jax.ShapeDtypeStruct((NP, H), jnp.float32),
            jax.ShapeDtypeStruct((NP, 1), jnp.float32),
        ],
    )(degp, graw)


def _tc_mid_body(p_ref, g_ref, dis_ref, b_ref, w_ref, out_ref):
    p = p_ref[...]
    dis = dis_ref[...]
    h = jnp.maximum(dis * (p[0] + p[1] + g_ref[...]) + b_ref[...], 0.0)
    out_ref[...] = jnp.dot(h, w_ref[...],
                           preferred_element_type=jnp.float32) * dis


def _make_tc_mid(h_in, h_out):
    @jax.jit
    def tc_mid(partials, g, dis, b2d, W):
        return pl.pallas_call(
            _tc_mid_body,
            grid=(NP // _R,),
            in_specs=[
                pl.BlockSpec((NC, _R, h_in), lambda i: (0, i, 0)),
                pl.BlockSpec((_R, h_in), lambda i: (i, 0)),
                pl.BlockSpec((_R, 1), lambda i: (i, 0)),
                pl.BlockSpec((1, h_in), lambda i: (0, 0)),
                pl.BlockSpec((h_in, h_out), lambda i: (0, 0)),
            ],
            out_specs=pl.BlockSpec((_R, h_out), lambda i: (i, 0)),
            out_shape=jax.ShapeDtypeStruct((NP, h_out), jnp.float32),
        )(partials, g, dis, b2d, W)
    return tc_mid


_tc2 = _make_tc_mid(H, H)


def _tc3_body(p_ref, g_ref, dis_ref, b_ref, g3_ref):
    p = p_ref[...]
    dis = dis_ref[...]
    g3_ref[...] = jnp.maximum(dis * (p[0] + p[1] + g_ref[...]) + b_ref[...],
                              0.0) * dis


@jax.jit
def _tc3(partials, g, dis, b2d):
    return pl.pallas_call(
        _tc3_body,
        grid=(NP // _R,),
        in_specs=[
            pl.BlockSpec((NC, _R, H), lambda i: (0, i, 0)),
            pl.BlockSpec((_R, H), lambda i: (i, 0)),
            pl.BlockSpec((_R, 1), lambda i: (i, 0)),
            pl.BlockSpec((1, H), lambda i: (0, 0)),
        ],
        out_specs=pl.BlockSpec((_R, H), lambda i: (i, 0)),
        out_shape=jax.ShapeDtypeStruct((NP, H), jnp.float32),
    )(partials, g, dis, b2d)


def _tc4_body(p_ref, g_ref, dis_ref, w_ref, b_ref, out_ref):
    p = p_ref[...]
    agg = dis_ref[...] * (p[0] + p[1] + g_ref[...])
    out_ref[...] = jnp.dot(agg, w_ref[...],
                           preferred_element_type=jnp.float32) + b_ref[...]


_R4 = 2000


@jax.jit
def _tc4(partials, g, dis, W3, b2d):
    return pl.pallas_call(
        _tc4_body,
        grid=(N // _R4,),
        in_specs=[
            pl.BlockSpec((NC, _R4, H), lambda i: (0, i, 0)),
            pl.BlockSpec((_R4, H), lambda i: (i, 0)),
            pl.BlockSpec((_R4, 1), lambda i: (i, 0)),
            pl.BlockSpec((H, D_OUT), lambda i: (0, 0)),
            pl.BlockSpec((1, D_OUT), lambda i: (0, 0)),
        ],
        out_specs=pl.BlockSpec((_R4, D_OUT), lambda i: (i, 0)),
        out_shape=jax.ShapeDtypeStruct((N, D_OUT), jnp.float32),
    )(partials, g, dis, W3, b2d)


def kernel(x, edge_index, W1, b1, W2, b2, W3, b3):
    src2 = edge_index[0].reshape(CHN, CH)
    dst2 = edge_index[1].reshape(CHN, CH)
    ones16 = jnp.ones((CH, 16), jnp.float32)
    z16 = jnp.zeros((NP, 16), jnp.float32)
    z64 = jnp.zeros((NP, H), jnp.float32)
    xp = jnp.pad(x, ((0, NP - N), (0, 0)))

    degp = _deg_partials(dst2, ones16, z16)
    g1raw = _tc1a(xp, W1)
    g1, dis = _tc1b(degp, g1raw)
    p1 = _mp64(g1, src2, dst2, z64)
    g2 = _tc2(p1, g1, dis, b1.reshape(1, H), W2)
    p2 = _mp64(g2, src2, dst2, z64)
    g3 = _tc3(p2, g2, dis, b2.reshape(1, H))
    p3 = _mp64(g3, src2, dst2, z64)
    return _tc4(p3, g3, dis, W3, b3.reshape(1, D_OUT))


# speedup vs baseline: 1.0497x; 1.0497x over previous
import functools

import jax
import jax.numpy as jnp
from jax import lax
from jax.experimental import pallas as pl
from jax.experimental.pallas import tpu as pltpu
from jax.experimental.pallas import tpu_sc as plsc

N = 10000
NP = 10240
E = 320000
D_IN = 128
H = 64
D_OUT = 128

NC = 2
NS = 16
NW = NC * NS
CH = 128
CHN = E // CH
WCH = CHN // NW
XTRA = CHN - WCH * NW

RPS = NP // NS

_mesh = plsc.VectorSubcoreMesh(core_axis_name="c", subcore_axis_name="s")
_sc_params = pltpu.CompilerParams(use_tc_tiling_on_sc=False)


def _chunk_range(cid, sid):
    wid = sid * NC + cid
    base = wid * WCH + jnp.minimum(wid, XTRA)
    extra = wid < XTRA
    return base, extra


def _deg_body(dst_hbm, ones_hbm, zeros_hbm, out_hbm, idx_v, ones_v, acc):
    cid = lax.axis_index("c")
    sid = lax.axis_index("s")
    base, extra = _chunk_range(cid, sid)
    my = pl.ds(sid * RPS, RPS)
    pltpu.sync_copy(zeros_hbm.at[my], acc.at[my])
    pltpu.sync_copy(ones_hbm, ones_v)
    pltpu.sync_copy(dst_hbm.at[pl.ds(base, WCH)], idx_v.at[pl.ds(0, WCH)])

    @pl.when(extra)
    def _():
        pltpu.sync_copy(dst_hbm.at[pl.ds(base + WCH, 1)],
                        idx_v.at[pl.ds(WCH, 1)])

    plsc.subcore_barrier()

    @pl.loop(0, WCH)
    def _(ci):
        pltpu.sync_copy(ones_v, acc.at[idx_v.at[ci]], add=True)

    @pl.when(extra)
    def _():
        pltpu.sync_copy(ones_v, acc.at[idx_v.at[WCH]], add=True)

    plsc.subcore_barrier()
    pltpu.sync_copy(acc.at[my], out_hbm.at[cid, my])


@jax.jit
def _deg_partials(dst3, ones16, zeros16):
    k = pl.kernel(
        _deg_body,
        out_type=jax.ShapeDtypeStruct((NC, NP, 16), jnp.float32),
        mesh=_mesh,
        scratch_types=[
            pltpu.VMEM((WCH + 1, CH), jnp.int32),
            pltpu.VMEM((CH, 16), jnp.float32),
            pltpu.VMEM_SHARED((NP, 16), jnp.float32),
        ],
        compiler_params=_sc_params,
    )
    return k(dst3, ones16, zeros16)


def _mp_body(g_hbm, src_hbm, dst_hbm, zeros_hbm, out_hbm,
             sidx_v, didx_v, rows, acc, semg):
    cid = lax.axis_index("c")
    sid = lax.axis_index("s")
    base, extra = _chunk_range(cid, sid)
    my = pl.ds(sid * RPS, RPS)
    pltpu.sync_copy(zeros_hbm.at[my], acc.at[my])
    pltpu.sync_copy(src_hbm.at[pl.ds(base, WCH)], sidx_v.at[pl.ds(0, WCH)])
    pltpu.sync_copy(dst_hbm.at[pl.ds(base, WCH)], didx_v.at[pl.ds(0, WCH)])

    @pl.when(extra)
    def _():
        pltpu.sync_copy(src_hbm.at[pl.ds(base + WCH, 1)],
                        sidx_v.at[pl.ds(WCH, 1)])
        pltpu.sync_copy(dst_hbm.at[pl.ds(base + WCH, 1)],
                        didx_v.at[pl.ds(WCH, 1)])

    plsc.subcore_barrier()

    for j in range(6):
        pltpu.async_copy(g_hbm.at[sidx_v.at[j]], rows[j], semg[j])

    @pl.loop(0, WCH - 6, step=6)
    def _(c):
        for j in range(6):
            pltpu.make_async_copy(g_hbm.at[sidx_v.at[c + j]],
                                  rows[j], semg[j]).wait()
            pltpu.sync_copy(rows[j], acc.at[didx_v.at[c + j]], add=True)
            pltpu.async_copy(g_hbm.at[sidx_v.at[c + 6 + j]], rows[j], semg[j])

    for j in range(6):
        pltpu.make_async_copy(g_hbm.at[sidx_v.at[WCH - 6 + j]],
                              rows[j], semg[j]).wait()
        pltpu.sync_copy(rows[j], acc.at[didx_v.at[WCH - 6 + j]], add=True)

    @pl.when(extra)
    def _():
        pltpu.async_copy(g_hbm.at[sidx_v.at[WCH]], rows[0], semg[0]).wait()
        pltpu.sync_copy(rows[0], acc.at[didx_v.at[WCH]], add=True)

    plsc.subcore_barrier()
    pltpu.sync_copy(acc.at[my], out_hbm.at[cid, my])


def _make_mp(h):
    @jax.jit
    def mp(g, src2, dst2, zeros):
        k = pl.kernel(
            _mp_body,
            out_type=jax.ShapeDtypeStruct((NC, NP, h), jnp.float32),
            mesh=_mesh,
            scratch_types=[
                pltpu.VMEM((WCH + 1, CH), jnp.int32),
                pltpu.VMEM((WCH + 1, CH), jnp.int32),
                [pltpu.VMEM((CH, h), jnp.float32) for _ in range(6)],
                pltpu.VMEM_SHARED((NP, h), jnp.float32),
                [pltpu.SemaphoreType.DMA for _ in range(6)],
            ],
            compiler_params=_sc_params,
        )
        return k(g, src2, dst2, zeros)
    return mp


_mp64 = _make_mp(H)


_R = 2048


def _tc1a_body(x_ref, w_ref, g_ref):
    g_ref[...] = jnp.dot(x_ref[...], w_ref[...],
                         preferred_element_type=jnp.float32)


@jax.jit
def _tc1a(x, W1):
    return pl.pallas_call(
        _tc1a_body,
        grid=(NP // _R,),
        in_specs=[
            pl.BlockSpec((_R, D_IN), lambda i: (i, 0)),
            pl.BlockSpec((D_IN, H), lambda i: (0, 0)),
        ],
        out_specs=pl.BlockSpec((_R, H), lambda i: (i, 0)),
        out_shape=jax.ShapeDtypeStruct((NP, H), jnp.float32),
    )(x, W1)


def _tc1b_body(p_ref, graw_ref, g_ref, dis_ref):
    p = p_ref[...]
    deg = p[0, :, 0:1] + p[1, :, 0:1] + 1.0
    dis = lax.rsqrt(deg)
    g_ref[...] = graw_ref[...] * dis
    dis_ref[...] = dis


@jax.jit
def _tc1b(degp, graw):
    return pl.pallas_call(
        _tc1b_body,
        grid=(NP // _R,),
        in_specs=[
            pl.BlockSpec((NC, _R, 16), lambda i: (0, i, 0)),
            pl.BlockSpec((_R, H), lambda i: (i, 0)),
        ],
        out_specs=[
            pl.BlockSpec((_R, H), lambda i: (i, 0)),
            pl.BlockSpec((_R, 1), lambda i: (i, 0)),
        ],
        out_shape=[
            jax.ShapeDtypeStruct((NP, H), jnp.float32),
            jax.ShapeDtypeStruct((NP, 1), jnp.float32),
        ],
    )(degp, graw)


def _tc_mid_body(p_ref, g_ref, dis_ref, b_ref, w_ref, out_ref):
    p = p_ref[...]
    dis = dis_ref[...]
    h = jnp.maximum(dis * (p[0] + p[1] + g_ref[...]) + b_ref[...], 0.0)
    out_ref[...] = jnp.dot(h, w_ref[...],
                           preferred_element_type=jnp.float32) * dis


def _make_tc_mid(h_in, h_out):
    @jax.jit
    def tc_mid(partials, g, dis, b2d, W):
        return pl.pallas_call(
            _tc_mid_body,
            grid=(NP // _R,),
            in_specs=[
                pl.BlockSpec((NC, _R, h_in), lambda i: (0, i, 0)),
                pl.BlockSpec((_R, h_in), lambda i: (i, 0)),
                pl.BlockSpec((_R, 1), lambda i: (i, 0)),
                pl.BlockSpec((1, h_in), lambda i: (0, 0)),
                pl.BlockSpec((h_in, h_out), lambda i: (0, 0)),
            ],
            out_specs=pl.BlockSpec((_R, h_out), lambda i: (i, 0)),
            out_shape=jax.ShapeDtypeStruct((NP, h_out), jnp.float32),
        )(partials, g, dis, b2d, W)
    return tc_mid


_tc2 = _make_tc_mid(H, H)


def _tc3_body(p_ref, g_ref, dis_ref, b_ref, g3_ref):
    p = p_ref[...]
    dis = dis_ref[...]
    g3_ref[...] = jnp.maximum(dis * (p[0] + p[1] + g_ref[...]) + b_ref[...],
                              0.0) * dis


@jax.jit
def _tc3(partials, g, dis, b2d):
    return pl.pallas_call(
        _tc3_body,
        grid=(NP // _R,),
        in_specs=[
            pl.BlockSpec((NC, _R, H), lambda i: (0, i, 0)),
            pl.BlockSpec((_R, H), lambda i: (i, 0)),
            pl.BlockSpec((_R, 1), lambda i: (i, 0)),
            pl.BlockSpec((1, H), lambda i: (0, 0)),
        ],
        out_specs=pl.BlockSpec((_R, H), lambda i: (i, 0)),
        out_shape=jax.ShapeDtypeStruct((NP, H), jnp.float32),
    )(partials, g, dis, b2d)


def _tc4_body(p_ref, g_ref, dis_ref, w_ref, b_ref, out_ref):
    p = p_ref[...]
    agg = dis_ref[...] * (p[0] + p[1] + g_ref[...])
    out_ref[...] = jnp.dot(agg, w_ref[...],
                           preferred_element_type=jnp.float32) + b_ref[...]


_R4 = 2000


@jax.jit
def _tc4(partials, g, dis, W3, b2d):
    return pl.pallas_call(
        _tc4_body,
        grid=(N // _R4,),
        in_specs=[
            pl.BlockSpec((NC, _R4, H), lambda i: (0, i, 0)),
            pl.BlockSpec((_R4, H), lambda i: (i, 0)),
            pl.BlockSpec((_R4, 1), lambda i: (i, 0)),
            pl.BlockSpec((H, D_OUT), lambda i: (0, 0)),
            pl.BlockSpec((1, D_OUT), lambda i: (0, 0)),
        ],
        out_specs=pl.BlockSpec((_R4, D_OUT), lambda i: (i, 0)),
        out_shape=jax.ShapeDtypeStruct((N, D_OUT), jnp.float32),
    )(partials, g, dis, W3, b2d)


def kernel(x, edge_index, W1, b1, W2, b2, W3, b3):
    src2 = edge_index[0].reshape(CHN, CH)
    dst2 = edge_index[1].reshape(CHN, CH)
    ones16 = jnp.ones((CH, 16), jnp.float32)
    z16 = jnp.zeros((NP, 16), jnp.float32)
    z64 = jnp.zeros((NP, H), jnp.float32)
    xp = jnp.pad(x, ((0, NP - N), (0, 0)))

    degp = _deg_partials(dst2, ones16, z16)
    g1raw = _tc1a(xp, W1)
    g1, dis = _tc1b(degp, g1raw)
    p1 = _mp64(g1, src2, dst2, z64)
    g2 = _tc2(p1, g1, dis, b1.reshape(1, H), W2)
    p2 = _mp64(g2, src2, dst2, z64)
    g3 = _tc3(p2, g2, dis, b2.reshape(1, H))
    p3 = _mp64(g3, src2, dst2, z64)
    return _tc4(p3, g3, dis, W3, b3.reshape(1, D_OUT))
